# baseline (speedup 1.0000x reference)
import numpy as _np

import jax
import jax.numpy as jnp
from jax import lax
from jax.experimental import pallas as pl
from jax.experimental.pallas import tpu as pltpu
from jax.experimental.pallas import tpu_sc as plsc

NC = 2
NS = 16
NW = NC * NS
LANES = 16

N_U = 5000
N_I = 5000
D = 128
E = 320000

NP = 5008
RPT = NP // NS
HN = 5120
EPT = E // NW
CHUNK = 125
CPT = EPT // CHUNK
VPT = EPT // LANES


_colperm = _np.zeros((D,), _np.int32)
for _k in range(D // 32):
    for _t in range(16):
        _colperm[_k * 32 + 2 * _t] = _k * 32 + _t
        _colperm[_k * 32 + 2 * _t + 1] = _k * 32 + 16 + _t


_mesh = plsc.VectorSubcoreMesh(
    core_axis_name="c", subcore_axis_name="s", num_cores=NC, num_subcores=NS
)

_sc_params = pltpu.CompilerParams(
    use_tc_tiling_on_sc=False, needs_layout_passes=False
)


def _hist_body(src_hbm, dst_hbm, hist_hbm, idx_v, sub_v, deg_v):
    c = lax.axis_index("c")
    s = lax.axis_index("s")
    wid = c * NS + s
    lane = lax.broadcasted_iota(jnp.int32, (LANES,), 0)
    ones = jnp.ones((LANES,), jnp.float32)
    zeros = jnp.zeros((LANES,), jnp.float32)

    for d, ref in ((0, src_hbm), (1, dst_hbm)):
        pltpu.sync_copy(ref.at[wid], idx_v)

        def zero_body(t, _):
            for u in range(8):
                sub_v[pl.ds((t * 8 + u) * LANES, LANES)] = zeros
            return _

        lax.fori_loop(0, NS * (HN // LANES) // 8, zero_body, 0)

        def edge_body(t, _):
            for u in range(5):
                idx = idx_v[pl.ds((t * 5 + u) * LANES, LANES)]
                plsc.addupdate_scatter(sub_v, [lane * HN + idx], ones)
            return _

        lax.fori_loop(0, VPT // 5, edge_body, 0)

        def red_body(i, _):
            acc = sub_v[pl.ds(i * LANES, LANES)]
            for r in range(1, NS):
                acc = acc + sub_v[pl.ds(r * HN + i * LANES, LANES)]
            deg_v[pl.ds(d * HN + i * LANES, LANES)] = acc
            return _

        lax.fori_loop(0, HN // LANES, red_body, 0)

    pltpu.sync_copy(deg_v.at[pl.ds(0, HN)], hist_hbm.at[wid])
    pltpu.sync_copy(deg_v.at[pl.ds(HN, HN)], hist_hbm.at[NW + wid])


_hist_call = pl.kernel(
    _hist_body,
    out_type=jax.ShapeDtypeStruct((2 * NW, HN), jnp.float32),
    mesh=_mesh,
    scratch_types=[
        pltpu.VMEM((EPT,), jnp.int32),
        pltpu.VMEM((NS * HN,), jnp.float32),
        pltpu.VMEM((2 * HN,), jnp.float32),
    ],
    compiler_params=_sc_params,
)


def _prep_body(hist_ref, u_ref, i_ref, wu_ref, wi_ref, inv_ref):
    h = hist_ref[...]
    r = lax.broadcasted_iota(jnp.int32, (2 * NW, 2), 0)
    col = lax.broadcasted_iota(jnp.int32, (2 * NW, 2), 1)
    sel = jnp.where((r < NW) == (col == 0), 1.0, 0.0).astype(jnp.float32)
    deg2 = lax.dot_general(
        h, sel, (((0,), (0,)), ((), ())), preferred_element_type=jnp.float32
    )
    inv2 = jnp.where(deg2 > 0, lax.rsqrt(jnp.maximum(deg2, 1.0)), 0.0)
    inv_ref[...] = inv2
    wu_ref[...] = (u_ref[...] * inv2[:NP, 0:1]).astype(jnp.bfloat16)
    wi_ref[...] = (i_ref[...] * inv2[:NP, 1:2]).astype(jnp.bfloat16)


_prep_call = pl.pallas_call(
    _prep_body,
    out_shape=[
        jax.ShapeDtypeStruct((NP, D), jnp.bfloat16),
        jax.ShapeDtypeStruct((NP, D), jnp.bfloat16),
        jax.ShapeDtypeStruct((HN, 2), jnp.float32),
    ],
)


def _main_body(
    wu_hbm, wi_hbm, src_hbm, dst_hbm, oi_hbm, ou_hbm,
    srcv, dstv, gb0, gb1, gb2, gb3, sb0, sb1, acc,
    gs0, gs1, gs2, gs3, ss0, ss1,
):
    c = lax.axis_index("c")
    s = lax.axis_index("s")
    wid = c * NS + s
    pltpu.sync_copy(src_hbm.at[wid], srcv)
    pltpu.sync_copy(dst_hbm.at[wid], dstv)

    zeros = jnp.zeros((LANES,), jnp.float32)

    def zero_buf():
        def zero_body(r, _):
            for k in range(D // LANES):
                sb0[r, pl.ds(k * LANES, LANES)] = zeros
            return _

        lax.fori_loop(0, CHUNK, zero_body, 0)

    row0 = s * RPT
    tail = RPT - 2 * CHUNK

    def zero_acc():
        pltpu.sync_copy(sb0, acc.at[pl.ds(row0, CHUNK)])
        pltpu.sync_copy(sb0, acc.at[pl.ds(row0 + CHUNK, CHUNK)])
        pltpu.sync_copy(sb0.at[pl.ds(0, tail)], acc.at[pl.ds(row0 + 2 * CHUNK, tail)])

    off = c * NP + row0

    gbufs = (gb0, gb1, gb2, gb3)
    sbufs = (sb0, sb1)
    gsems = (gs0, gs1, gs2, gs3)
    ssems = (ss0, ss1)

    def run_pass(table_hbm, gidx, sidx):
        def body(j4, _):
            j = 4 * j4
            gds = [
                pltpu.async_copy(table_hbm.at[gidx.at[j + u]], gbufs[u], gsems[u])
                for u in range(4)
            ]
            sds = []
            for u in range(4):
                gds[u].wait()
                if u >= 2:
                    sds[u - 2].wait()
                gb = gbufs[u]
                sb = sbufs[u % 2]

                def conv(r, _):
                    for k in range(D // 32):
                        x = gb[r, pl.ds(32 * k, 32)]
                        a, b = plsc.unpack(
                            x,
                            format=plsc.PackFormat.INTERLEAVED,
                            preferred_element_type=jnp.float32,
                        )
                        sb[r, pl.ds(32 * k, LANES)] = a
                        sb[r, pl.ds(32 * k + LANES, LANES)] = b
                    return _

                lax.fori_loop(0, CHUNK, conv, 0)
                sds.append(
                    pltpu.async_copy(sb, acc.at[sidx.at[j + u]], ssems[u % 2], add=True)
                )
            sds[2].wait()
            sds[3].wait()
            return _

        lax.fori_loop(0, CPT // 4, body, 0)

    zero_buf()
    zero_acc()
    plsc.subcore_barrier()
    run_pass(wu_hbm, srcv, dstv)
    plsc.subcore_barrier()
    pltpu.sync_copy(acc.at[pl.ds(row0, RPT)], oi_hbm.at[pl.ds(off, RPT)])

    zero_buf()
    zero_acc()
    plsc.subcore_barrier()
    run_pass(wi_hbm, dstv, srcv)
    plsc.subcore_barrier()
    pltpu.sync_copy(acc.at[pl.ds(row0, RPT)], ou_hbm.at[pl.ds(off, RPT)])


_main_call = pl.kernel(
    _main_body,
    out_type=[
        jax.ShapeDtypeStruct((NC * NP, D), jnp.float32),
        jax.ShapeDtypeStruct((NC * NP, D), jnp.float32),
    ],
    mesh=_mesh,
    scratch_types=[
        pltpu.VMEM((CPT, CHUNK), jnp.int32),
        pltpu.VMEM((CPT, CHUNK), jnp.int32),
        pltpu.VMEM((CHUNK, D), jnp.bfloat16),
        pltpu.VMEM((CHUNK, D), jnp.bfloat16),
        pltpu.VMEM((CHUNK, D), jnp.bfloat16),
        pltpu.VMEM((CHUNK, D), jnp.bfloat16),
        pltpu.VMEM((CHUNK, D), jnp.float32),
        pltpu.VMEM((CHUNK, D), jnp.float32),
        pltpu.VMEM_SHARED((NP, D), jnp.float32),
        pltpu.SemaphoreType.DMA,
        pltpu.SemaphoreType.DMA,
        pltpu.SemaphoreType.DMA,
        pltpu.SemaphoreType.DMA,
        pltpu.SemaphoreType.DMA,
        pltpu.SemaphoreType.DMA,
    ],
    compiler_params=_sc_params,
)


def _fin_body(oi_ref, ou_ref, inv_ref, items_ref, users_ref):
    inv2 = inv_ref[...]
    items_ref[...] = (oi_ref[0:NP, :] + oi_ref[NP : 2 * NP, :]) * inv2[:NP, 1:2]
    users_ref[...] = (ou_ref[0:NP, :] + ou_ref[NP : 2 * NP, :]) * inv2[:NP, 0:1]


_fin_call = pl.pallas_call(
    _fin_body,
    out_shape=[
        jax.ShapeDtypeStruct((NP, D), jnp.float32),
        jax.ShapeDtypeStruct((NP, D), jnp.float32),
    ],
)


@jax.jit
def kernel(ufeats, ifeats, edge_index):
    src = edge_index[0].astype(jnp.int32)
    dst = edge_index[1].astype(jnp.int32)
    src3 = src.reshape(NW, CPT, CHUNK)
    dst3 = dst.reshape(NW, CPT, CHUNK)
    srcf = src.reshape(NW, EPT)
    dstf = dst.reshape(NW, EPT)
    zrows = jnp.zeros((NP - N_U, D), jnp.float32)
    colperm = jnp.asarray(_colperm)
    up = jnp.concatenate([ufeats, zrows], axis=0)[:, colperm]
    ip = jnp.concatenate([ifeats, zrows], axis=0)[:, colperm]

    hist = _hist_call(srcf, dstf)
    wu, wi, inv2 = _prep_call(hist, up, ip)
    oi, ou = _main_call(wu, wi, src3, dst3)
    items, users = _fin_call(oi, ou, inv2)
    return users[:N_U], items[:N_I]


# speedup vs baseline: 1.6615x; 1.6615x over previous
import numpy as _np

import jax
import jax.numpy as jnp
from jax import lax
from jax.experimental import pallas as pl
from jax.experimental.pallas import tpu as pltpu
from jax.experimental.pallas import tpu_sc as plsc

NC = 2
NS = 16
NW = NC * NS
LANES = 16

N_U = 5000
N_I = 5000
D = 128
E = 320000

NP = 5008
RPT = NP // NS
HN = 5120
EPT = E // NW
CHUNK = 125
CPT = EPT // CHUNK
VPT = EPT // LANES


_mesh = plsc.VectorSubcoreMesh(
    core_axis_name="c", subcore_axis_name="s", num_cores=NC, num_subcores=NS
)

_sc_params = pltpu.CompilerParams(
    use_tc_tiling_on_sc=False, needs_layout_passes=False
)


def _hist_body(src_hbm, dst_hbm, hist_hbm, idx_v, sub_v, deg_v):
    c = lax.axis_index("c")
    s = lax.axis_index("s")
    wid = c * NS + s
    lane = lax.broadcasted_iota(jnp.int32, (LANES,), 0)
    ones = jnp.ones((LANES,), jnp.float32)
    zeros = jnp.zeros((LANES,), jnp.float32)

    for d, ref in ((0, src_hbm), (1, dst_hbm)):
        pltpu.sync_copy(ref.at[wid], idx_v)

        def zero_body(t, _):
            for u in range(8):
                sub_v[pl.ds((t * 8 + u) * LANES, LANES)] = zeros
            return _

        lax.fori_loop(0, NS * (HN // LANES) // 8, zero_body, 0)

        def edge_body(t, _):
            for u in range(5):
                idx = idx_v[pl.ds((t * 5 + u) * LANES, LANES)]
                plsc.addupdate_scatter(sub_v, [lane * HN + idx], ones)
            return _

        lax.fori_loop(0, VPT // 5, edge_body, 0)

        def red_body(i, _):
            acc = sub_v[pl.ds(i * LANES, LANES)]
            for r in range(1, NS):
                acc = acc + sub_v[pl.ds(r * HN + i * LANES, LANES)]
            deg_v[pl.ds(d * HN + i * LANES, LANES)] = acc
            return _

        lax.fori_loop(0, HN // LANES, red_body, 0)

    pltpu.sync_copy(deg_v.at[pl.ds(0, HN)], hist_hbm.at[wid])
    pltpu.sync_copy(deg_v.at[pl.ds(HN, HN)], hist_hbm.at[NW + wid])


_hist_call = pl.kernel(
    _hist_body,
    out_type=jax.ShapeDtypeStruct((2 * NW, HN), jnp.float32),
    mesh=_mesh,
    scratch_types=[
        pltpu.VMEM((EPT,), jnp.int32),
        pltpu.VMEM((NS * HN,), jnp.float32),
        pltpu.VMEM((2 * HN,), jnp.float32),
    ],
    compiler_params=_sc_params,
)


def _prep_body(hist_ref, u_ref, i_ref, wu_ref, wi_ref, inv_ref):
    h = hist_ref[...]
    r = lax.broadcasted_iota(jnp.int32, (2 * NW, 2), 0)
    col = lax.broadcasted_iota(jnp.int32, (2 * NW, 2), 1)
    sel = jnp.where((r < NW) == (col == 0), 1.0, 0.0).astype(jnp.float32)
    deg2 = lax.dot_general(
        h, sel, (((0,), (0,)), ((), ())), preferred_element_type=jnp.float32
    )
    inv2 = jnp.where(deg2 > 0, lax.rsqrt(jnp.maximum(deg2, 1.0)), 0.0)
    inv_ref[...] = inv2
    wu_ref[0:N_U, :] = u_ref[...] * inv2[:N_U, 0:1]
    wu_ref[N_U:NP, :] = jnp.zeros((NP - N_U, D), jnp.float32)
    wi_ref[0:N_I, :] = i_ref[...] * inv2[:N_I, 1:2]
    wi_ref[N_I:NP, :] = jnp.zeros((NP - N_I, D), jnp.float32)


_prep_call = pl.pallas_call(
    _prep_body,
    out_shape=[
        jax.ShapeDtypeStruct((NP, D), jnp.float32),
        jax.ShapeDtypeStruct((NP, D), jnp.float32),
        jax.ShapeDtypeStruct((HN, 2), jnp.float32),
    ],
)


def _main_body(
    wu_hbm, wi_hbm, src_hbm, dst_hbm, oi_hbm, ou_hbm,
    srcv, dstv, bufu, bufi, bufc, bufd, acc, sem_u, sem_i, sem_c, sem_d,
):
    c = lax.axis_index("c")
    s = lax.axis_index("s")
    wid = c * NS + s
    pltpu.sync_copy(src_hbm.at[wid], srcv)
    pltpu.sync_copy(dst_hbm.at[wid], dstv)

    zeros = jnp.zeros((LANES,), jnp.float32)

    def zero_buf():
        def zero_body(r, _):
            for k in range(D // LANES):
                bufu[r, pl.ds(k * LANES, LANES)] = zeros
            return _

        lax.fori_loop(0, CHUNK, zero_body, 0)

    row0 = s * RPT
    tail = RPT - 2 * CHUNK

    def zero_acc():
        pltpu.sync_copy(bufu, acc.at[pl.ds(row0, CHUNK)])
        pltpu.sync_copy(bufu, acc.at[pl.ds(row0 + CHUNK, CHUNK)])
        pltpu.sync_copy(bufu.at[pl.ds(0, tail)], acc.at[pl.ds(row0 + 2 * CHUNK, tail)])

    off = c * NP + row0

    bufs = (bufu, bufi, bufc, bufd)
    sems = (sem_u, sem_i, sem_c, sem_d)

    def run_pass(table_hbm, gidx, sidx):
        def body(j4, _):
            j = 4 * j4
            ds_ = [
                pltpu.async_copy(table_hbm.at[gidx.at[j + u]], bufs[u], sems[u])
                for u in range(4)
            ]
            for u in range(4):
                ds_[u].wait()
                pltpu.sync_copy(bufs[u], acc.at[sidx.at[j + u]], add=True)
            return _

        lax.fori_loop(0, CPT // 4, body, 0)

    zero_buf()
    zero_acc()
    plsc.subcore_barrier()
    run_pass(wu_hbm, srcv, dstv)
    plsc.subcore_barrier()
    pltpu.sync_copy(acc.at[pl.ds(row0, RPT)], oi_hbm.at[pl.ds(off, RPT)])

    zero_buf()
    zero_acc()
    plsc.subcore_barrier()
    run_pass(wi_hbm, dstv, srcv)
    plsc.subcore_barrier()
    pltpu.sync_copy(acc.at[pl.ds(row0, RPT)], ou_hbm.at[pl.ds(off, RPT)])


_main_call = pl.kernel(
    _main_body,
    out_type=[
        jax.ShapeDtypeStruct((NC * NP, D), jnp.float32),
        jax.ShapeDtypeStruct((NC * NP, D), jnp.float32),
    ],
    mesh=_mesh,
    scratch_types=[
        pltpu.VMEM((CPT, CHUNK), jnp.int32),
        pltpu.VMEM((CPT, CHUNK), jnp.int32),
        pltpu.VMEM((CHUNK, D), jnp.float32),
        pltpu.VMEM((CHUNK, D), jnp.float32),
        pltpu.VMEM((CHUNK, D), jnp.float32),
        pltpu.VMEM((CHUNK, D), jnp.float32),
        pltpu.VMEM_SHARED((NP, D), jnp.float32),
        pltpu.SemaphoreType.DMA,
        pltpu.SemaphoreType.DMA,
        pltpu.SemaphoreType.DMA,
        pltpu.SemaphoreType.DMA,
    ],
    compiler_params=_sc_params,
)


def _fin_body(oi_ref, ou_ref, inv_ref, items_ref, users_ref):
    inv2 = inv_ref[...]
    items_ref[...] = (oi_ref[0:N_I, :] + oi_ref[NP : NP + N_I, :]) * inv2[:N_I, 1:2]
    users_ref[...] = (ou_ref[0:N_U, :] + ou_ref[NP : NP + N_U, :]) * inv2[:N_U, 0:1]


_fin_call = pl.pallas_call(
    _fin_body,
    out_shape=[
        jax.ShapeDtypeStruct((N_I, D), jnp.float32),
        jax.ShapeDtypeStruct((N_U, D), jnp.float32),
    ],
)


@jax.jit
def kernel(ufeats, ifeats, edge_index):
    src = edge_index[0].astype(jnp.int32)
    dst = edge_index[1].astype(jnp.int32)
    src3 = src.reshape(NW, CPT, CHUNK)
    dst3 = dst.reshape(NW, CPT, CHUNK)
    srcf = src.reshape(NW, EPT)
    dstf = dst.reshape(NW, EPT)

    hist = _hist_call(srcf, dstf)
    wu, wi, inv2 = _prep_call(hist, ufeats, ifeats)
    oi, ou = _main_call(wu, wi, src3, dst3)
    items, users = _fin_call(oi, ou, inv2)
    return users, items
